# baseline (speedup 1.0000x reference)
import jax
import jax.numpy as jnp
from jax.experimental import pallas as pl
from jax.experimental.pallas import tpu as pltpu

B, N, N0, D = 64, 4096, 1024, 128


def _tc_body(idx_ref, tbl_ref, out_ref):
    idx = idx_ref[0, 0, :]
    tbl = tbl_ref[0].astype(jnp.bfloat16)
    iota = jax.lax.broadcasted_iota(jnp.int32, (N0, N), 1)
    onehot = (idx[:, None] == iota).astype(jnp.bfloat16)
    out_ref[0] = jnp.dot(onehot, tbl, preferred_element_type=jnp.float32)


@jax.jit
def _tc_gather(table, idx):
    idx3 = idx.reshape(B, 1, N0)
    return pl.pallas_call(
        _tc_body,
        grid=(B,),
        in_specs=[
            pl.BlockSpec((1, 1, N0), lambda b: (b, 0, 0)),
            pl.BlockSpec((1, N, D), lambda b: (b, 0, 0)),
        ],
        out_specs=pl.BlockSpec((1, N0, D), lambda b: (b, 0, 0)),
        out_shape=jax.ShapeDtypeStruct((B, N0, D), jnp.float32),
    )(idx3, table)


def kernel(atom_fea, target_index):
    return _tc_gather(atom_fea, target_index.astype(jnp.int32))


# speedup vs baseline: 3.5558x; 3.5558x over previous
import jax
import jax.numpy as jnp
from jax import lax
from jax.experimental import pallas as pl
from jax.experimental.pallas import tpu as pltpu
from jax.experimental.pallas import tpu_sc as plsc

B = 64
N = 4096
N0 = 1024
D = 128

NC = 2
NS = 16
NW = NC * NS

BATCH_PER_W = B // NW
CHUNK = 128
CHUNKS_PER_BATCH = N0 // CHUNK
NCHUNK = BATCH_PER_W * CHUNKS_PER_BATCH

NBUF = 4
G = 2
NGROUP = NCHUNK // NBUF


def _body(table_hbm, idx_hbm, out_hbm, idx_v, *rest):
    bufs = rest[:NBUF]
    gsems = rest[NBUF:2 * NBUF]
    wsems = rest[2 * NBUF:3 * NBUF]

    cid = lax.axis_index("c")
    sid = lax.axis_index("s")
    wid = sid * NC + cid
    base_batch = wid * BATCH_PER_W

    pltpu.sync_copy(idx_hbm.at[pl.ds(base_batch, BATCH_PER_W)], idx_v)

    def out_slice(c):
        b = c // CHUNKS_PER_BATCH
        col = (c % CHUNKS_PER_BATCH) * CHUNK
        return out_hbm.at[base_batch + b].at[pl.ds(col, CHUNK)]

    def start_gather(c, b):
        bb = c // CHUNKS_PER_BATCH
        col = (c % CHUNKS_PER_BATCH) * CHUNK
        pltpu.async_copy(
            table_hbm.at[base_batch + bb].at[idx_v.at[bb, pl.ds(col, CHUNK)]],
            bufs[b], gsems[b])

    def wait_gather(b):
        pltpu.make_async_copy(
            table_hbm.at[0].at[pl.ds(0, CHUNK)], bufs[b], gsems[b]).wait()

    def start_write(c, b):
        pltpu.async_copy(bufs[b], out_slice(c), wsems[b])

    def wait_write(b):
        pltpu.make_async_copy(
            bufs[b], out_hbm.at[0].at[pl.ds(0, CHUNK)], wsems[b]).wait()

    for b in range(G):
        start_gather(b, b)

    def group(g, _):
        for b in range(NBUF):
            c = g * NBUF + b
            wait_gather(b)
            start_write(c, b)
            f = c + G
            bf = (b + G) % NBUF

            @pl.when(f < NCHUNK)
            def _():
                @pl.when(f >= NBUF)
                def _():
                    wait_write(bf)
                start_gather(f, bf)
        return _

    lax.fori_loop(0, NGROUP, group, None, unroll=False)

    for b in range(NBUF):
        wait_write(b)


@jax.jit
def _gather(table, idx):
    mesh = plsc.VectorSubcoreMesh(
        core_axis_name="c", subcore_axis_name="s",
        num_cores=NC, num_subcores=NS)
    return pl.kernel(
        _body,
        out_type=jax.ShapeDtypeStruct((B, N0, D), jnp.float32),
        mesh=mesh,
        scratch_types=(
            [pltpu.VMEM((BATCH_PER_W, N0), jnp.int32)]
            + [pltpu.VMEM((CHUNK, D), jnp.float32) for _ in range(NBUF)]
            + [pltpu.SemaphoreType.DMA for _ in range(2 * NBUF)]
        ),
    )(table, idx)


def kernel(atom_fea, target_index):
    return _gather(atom_fea, target_index.astype(jnp.int32))


# speedup vs baseline: 3.5616x; 1.0016x over previous
import jax
import jax.numpy as jnp
from jax import lax
from jax.experimental import pallas as pl
from jax.experimental.pallas import tpu as pltpu
from jax.experimental.pallas import tpu_sc as plsc

B = 64
N = 4096
N0 = 1024
D = 128

NC = 2
NS = 16
NW = NC * NS

BATCH_PER_W = B // NW
CHUNK = 128
CHUNKS_PER_BATCH = N0 // CHUNK
NCHUNK = BATCH_PER_W * CHUNKS_PER_BATCH

NBUF = 4
G = 2
NGROUP = NCHUNK // NBUF


def _body(table_hbm, idx_hbm, out_hbm, idx_v, *rest):
    bufs = rest[:NBUF]
    gsems = rest[NBUF:2 * NBUF]
    wsems = rest[2 * NBUF:3 * NBUF]
    isems = rest[3 * NBUF:3 * NBUF + BATCH_PER_W]

    cid = lax.axis_index("c")
    sid = lax.axis_index("s")
    wid = sid * NC + cid
    base_batch = wid * BATCH_PER_W

    idx_copies = [
        pltpu.async_copy(idx_hbm.at[pl.ds(base_batch + r, 1)],
                         idx_v.at[pl.ds(r, 1)], isems[r])
        for r in range(BATCH_PER_W)]
    idx_copies[0].wait()

    def out_slice(c):
        b = c // CHUNKS_PER_BATCH
        col = (c % CHUNKS_PER_BATCH) * CHUNK
        return out_hbm.at[base_batch + b].at[pl.ds(col, CHUNK)]

    def start_gather(c, b):
        bb = c // CHUNKS_PER_BATCH
        col = (c % CHUNKS_PER_BATCH) * CHUNK
        pltpu.async_copy(
            table_hbm.at[base_batch + bb].at[idx_v.at[bb, pl.ds(col, CHUNK)]],
            bufs[b], gsems[b])

    def wait_gather(b):
        pltpu.make_async_copy(
            table_hbm.at[0].at[pl.ds(0, CHUNK)], bufs[b], gsems[b]).wait()

    def start_write(c, b):
        pltpu.async_copy(bufs[b], out_slice(c), wsems[b])

    def wait_write(b):
        pltpu.make_async_copy(
            bufs[b], out_hbm.at[0].at[pl.ds(0, CHUNK)], wsems[b]).wait()

    for b in range(G):
        start_gather(b, b)
    for r in range(1, BATCH_PER_W):
        idx_copies[r].wait()

    def group(g, _):
        for b in range(NBUF):
            c = g * NBUF + b
            wait_gather(b)
            start_write(c, b)
            f = c + G
            bf = (b + G) % NBUF

            @pl.when(f < NCHUNK)
            def _():
                @pl.when(f >= NBUF)
                def _():
                    wait_write(bf)
                start_gather(f, bf)
        return _

    lax.fori_loop(0, NGROUP, group, None, unroll=False)

    for b in range(NBUF):
        wait_write(b)


@jax.jit
def _gather(table, idx):
    mesh = plsc.VectorSubcoreMesh(
        core_axis_name="c", subcore_axis_name="s",
        num_cores=NC, num_subcores=NS)
    return pl.kernel(
        _body,
        out_type=jax.ShapeDtypeStruct((B, N0, D), jnp.float32),
        mesh=mesh,
        scratch_types=(
            [pltpu.VMEM((BATCH_PER_W, N0), jnp.int32)]
            + [pltpu.VMEM((CHUNK, D), jnp.float32) for _ in range(NBUF)]
            + [pltpu.SemaphoreType.DMA for _ in range(2 * NBUF + BATCH_PER_W)]
        ),
    )(table, idx)


def kernel(atom_fea, target_index):
    return _gather(atom_fea, target_index.astype(jnp.int32))
